# baseline (speedup 1.0000x reference)
import jax
import jax.numpy as jnp
from jax.experimental import pallas as pl
from jax.experimental.pallas import tpu as pltpu

_B, _C, _H = 8, 384, 224
_R, _E, _K = 24, 8, 2
_HO = _H // 2
_NPIX = _HO * _HO
_CB = 32
_NCB = _C // _CB
_NF = _HO * _H


def _stage1_kernel(x_ref, w_ref, pw_ref, fwt_ref, fb_ref, gidx_ref,
                   wout_ref, iout_ref, acc_ref, pool_ref):
    b = pl.program_id(0)
    c = pl.program_id(1)

    @pl.when(c == 0)
    def _init():
        acc_ref[...] = jnp.zeros_like(acc_ref)

    xb = x_ref[0]
    w = w_ref[...]

    def wcol(k):
        return w[:, k][:, None].astype(jnp.bfloat16)

    x4 = xb.reshape(_CB, _H // 8, 8, _H)
    xp = jnp.take_along_axis(x4, gidx_ref[...], axis=2)
    ev = xp[:, :, 0:4, :].reshape(_CB, _NF).astype(jnp.bfloat16)
    od = xp[:, :, 4:8, :].reshape(_CB, _NF).astype(jnp.bfloat16)

    zrow = jnp.zeros((_CB, _H), jnp.bfloat16)
    zone = jnp.zeros((_CB, 1), jnp.bfloat16)
    up = jnp.concatenate([zrow, od[:, :-_H]], axis=1)

    a0 = wcol(0) * up + wcol(3) * ev + wcol(6) * od
    a1 = wcol(1) * up + wcol(4) * ev + wcol(7) * od
    a2 = wcol(2) * up + wcol(5) * ev + wcol(8) * od

    s = jax.lax.broadcasted_iota(jnp.int32, (1, _NF), 1) % _H
    m0 = (s != 0).astype(jnp.bfloat16)

    y = (m0 * jnp.concatenate([zone, a0[:, :-1]], axis=1)
         + a1
         + jnp.concatenate([a2[:, 1:], zone], axis=1)
         + wcol(9))
    y = y * jax.nn.sigmoid(y)
    acc_ref[...] += jnp.dot(pw_ref[0].astype(jnp.bfloat16), y,
                            preferred_element_type=jnp.float32)

    @pl.when(c == _NCB - 1)
    def _finish():
        z = acc_ref[...]
        z = z * jax.nn.sigmoid(z)
        p = jax.lax.broadcasted_iota(jnp.int32, (1, _NF), 1)
        pool = ((p % 2) == 0).astype(jnp.float32)
        pool_ref[b, :] = jnp.sum(z * pool, axis=1) * (1.0 / _NPIX)

    @pl.when((b == _B - 1) & (c == _NCB - 1))
    def _head():
        pp = pool_ref[...]
        logits = jnp.dot(pp, fwt_ref[...],
                         preferred_element_type=jnp.float32) + fb_ref[...]
        mx = jnp.max(logits, axis=1, keepdims=True)
        e = jnp.exp(logits - mx)
        probs = e / jnp.sum(e, axis=1, keepdims=True)
        colid = jax.lax.broadcasted_iota(jnp.int32, (_B, _E), 1)
        i1 = jnp.min(jnp.where(logits == mx, colid, _E), axis=1,
                     keepdims=True)
        masked = jnp.where(colid == i1, -jnp.inf, logits)
        v2 = jnp.max(masked, axis=1, keepdims=True)
        i2 = jnp.min(jnp.where(masked == v2, colid, _E), axis=1,
                     keepdims=True)
        s1 = jnp.sum(jnp.where(colid == i1, probs, 0.0), axis=1,
                     keepdims=True)
        s2 = jnp.sum(jnp.where(colid == i2, probs, 0.0), axis=1,
                     keepdims=True)
        tot = s1 + s2 + 1e-6
        wout_ref[...] = jnp.concatenate([s1 / tot, s2 / tot], axis=1)
        iout_ref[...] = jnp.concatenate([i1, i2], axis=1)


def kernel(x, dw_w, bn_gamma, bn_beta, bn_mean, bn_var, pw_w, fc_w, fc_b):
    scale = bn_gamma * jax.lax.rsqrt(bn_var + 1e-5)
    bias = bn_beta - bn_mean * scale
    w9 = dw_w.reshape(_C, 9) * scale[:, None]
    wpack = jnp.concatenate(
        [w9, bias[:, None], jnp.zeros((_C, 6), jnp.float32)], axis=1)
    pw = pw_w.reshape(_R, _NCB, _CB).transpose(1, 0, 2)

    call = pl.pallas_call(
        _stage1_kernel,
        grid=(_B, _NCB),
        in_specs=[
            pl.BlockSpec((1, _CB, _H, _H), lambda b, c: (b, c, 0, 0)),
            pl.BlockSpec((_CB, 16), lambda b, c: (c, 0)),
            pl.BlockSpec((1, _R, _CB), lambda b, c: (c, 0, 0)),
            pl.BlockSpec((_R, _E), lambda b, c: (0, 0)),
            pl.BlockSpec((1, _E), lambda b, c: (0, 0)),
            pl.BlockSpec((_CB, _H // 8, 8, _H), lambda b, c: (0, 0, 0, 0)),
        ],
        out_specs=(pl.BlockSpec((_B, _K), lambda b, c: (0, 0)),
                   pl.BlockSpec((_B, _K), lambda b, c: (0, 0))),
        out_shape=(jax.ShapeDtypeStruct((_B, _K), jnp.float32),
                   jax.ShapeDtypeStruct((_B, _K), jnp.int32)),
        scratch_shapes=[pltpu.VMEM((_R, _NF), jnp.float32),
                        pltpu.VMEM((_B, _R), jnp.float32)],
        compiler_params=pltpu.CompilerParams(
            dimension_semantics=("arbitrary", "arbitrary")),
    )
    u = jax.lax.broadcasted_iota(jnp.int32, (_CB, _H // 8, 8, _H), 2)
    gidx = ((u & 3) << 1) | (u >> 2)
    weights, topk_idx = call(x, wpack, pw, fc_w.T, fc_b.reshape(1, _E), gidx)
    return weights, topk_idx


# speedup vs baseline: 1.0890x; 1.0890x over previous
import jax
import jax.numpy as jnp
from jax.experimental import pallas as pl
from jax.experimental.pallas import tpu as pltpu

_B, _C, _H = 8, 384, 224
_R, _E, _K = 24, 8, 2
_HO = _H // 2
_NPIX = _HO * _HO
_CB = 32
_NCB = _C // _CB
_NF = _HO * _H


def _stage1_kernel(x_ref, w_ref, pw_ref, fwt_ref, fb_ref,
                   wout_ref, iout_ref, acc_ref, pool_ref):
    b = pl.program_id(0)
    c = pl.program_id(1)

    @pl.when(c == 0)
    def _init():
        acc_ref[...] = jnp.zeros_like(acc_ref)

    xb = x_ref[0]
    w = w_ref[...]

    def wcol(k):
        return w[:, k][:, None].astype(jnp.bfloat16)

    x4 = xb.reshape(_CB, _H // 8, 8, _H)
    u = jax.lax.broadcasted_iota(jnp.int32, (_CB, _H // 8, 8, _H), 2)
    gidx = ((u & 3) << 1) | (u >> 2)
    xp = jnp.take_along_axis(x4, gidx, axis=2)
    ev = xp[:, :, 0:4, :].reshape(_CB, _NF).astype(jnp.bfloat16)
    od = xp[:, :, 4:8, :].reshape(_CB, _NF).astype(jnp.bfloat16)

    zrow = jnp.zeros((_CB, _H), jnp.bfloat16)
    zone = jnp.zeros((_CB, 1), jnp.bfloat16)
    up = jnp.concatenate([zrow, od[:, :-_H]], axis=1)

    a0 = wcol(0) * up + wcol(3) * ev + wcol(6) * od
    a1 = wcol(1) * up + wcol(4) * ev + wcol(7) * od
    a2 = wcol(2) * up + wcol(5) * ev + wcol(8) * od

    s = jax.lax.broadcasted_iota(jnp.int32, (1, _NF), 1) % _H
    m0 = (s != 0).astype(jnp.bfloat16)

    y = (m0 * jnp.concatenate([zone, a0[:, :-1]], axis=1)
         + a1
         + jnp.concatenate([a2[:, 1:], zone], axis=1)
         + wcol(9))
    y = y * jax.nn.sigmoid(y)
    acc_ref[...] += jnp.dot(pw_ref[0].astype(jnp.bfloat16), y,
                            preferred_element_type=jnp.float32)

    @pl.when(c == _NCB - 1)
    def _finish():
        z = acc_ref[...]
        z = z * jax.nn.sigmoid(z)
        p = jax.lax.broadcasted_iota(jnp.int32, (1, _NF), 1)
        pool = ((p % 2) == 0).astype(jnp.float32)
        pool_ref[b, :] = jnp.sum(z * pool, axis=1) * (1.0 / _NPIX)

    @pl.when((b == _B - 1) & (c == _NCB - 1))
    def _head():
        pp = pool_ref[...]
        logits = jnp.dot(pp, fwt_ref[...],
                         preferred_element_type=jnp.float32) + fb_ref[...]
        mx = jnp.max(logits, axis=1, keepdims=True)
        e = jnp.exp(logits - mx)
        probs = e / jnp.sum(e, axis=1, keepdims=True)
        colid = jax.lax.broadcasted_iota(jnp.int32, (_B, _E), 1)
        i1 = jnp.min(jnp.where(logits == mx, colid, _E), axis=1,
                     keepdims=True)
        masked = jnp.where(colid == i1, -jnp.inf, logits)
        v2 = jnp.max(masked, axis=1, keepdims=True)
        i2 = jnp.min(jnp.where(masked == v2, colid, _E), axis=1,
                     keepdims=True)
        s1 = jnp.sum(jnp.where(colid == i1, probs, 0.0), axis=1,
                     keepdims=True)
        s2 = jnp.sum(jnp.where(colid == i2, probs, 0.0), axis=1,
                     keepdims=True)
        tot = s1 + s2 + 1e-6
        wout_ref[...] = jnp.concatenate([s1 / tot, s2 / tot], axis=1)
        iout_ref[...] = jnp.concatenate([i1, i2], axis=1)


def kernel(x, dw_w, bn_gamma, bn_beta, bn_mean, bn_var, pw_w, fc_w, fc_b):
    scale = bn_gamma * jax.lax.rsqrt(bn_var + 1e-5)
    bias = bn_beta - bn_mean * scale
    w9 = dw_w.reshape(_C, 9) * scale[:, None]
    wpack = jnp.concatenate(
        [w9, bias[:, None], jnp.zeros((_C, 6), jnp.float32)], axis=1)
    pw = pw_w.reshape(_R, _NCB, _CB).transpose(1, 0, 2)

    call = pl.pallas_call(
        _stage1_kernel,
        grid=(_B, _NCB),
        in_specs=[
            pl.BlockSpec((1, _CB, _H, _H), lambda b, c: (b, c, 0, 0)),
            pl.BlockSpec((_CB, 16), lambda b, c: (c, 0)),
            pl.BlockSpec((1, _R, _CB), lambda b, c: (c, 0, 0)),
            pl.BlockSpec((_R, _E), lambda b, c: (0, 0)),
            pl.BlockSpec((1, _E), lambda b, c: (0, 0)),
        ],
        out_specs=(pl.BlockSpec((_B, _K), lambda b, c: (0, 0)),
                   pl.BlockSpec((_B, _K), lambda b, c: (0, 0))),
        out_shape=(jax.ShapeDtypeStruct((_B, _K), jnp.float32),
                   jax.ShapeDtypeStruct((_B, _K), jnp.int32)),
        scratch_shapes=[pltpu.VMEM((_R, _NF), jnp.float32),
                        pltpu.VMEM((_B, _R), jnp.float32)],
        compiler_params=pltpu.CompilerParams(
            dimension_semantics=("arbitrary", "arbitrary")),
    )
    weights, topk_idx = call(x, wpack, pw, fc_w.T, fc_b.reshape(1, _E))
    return weights, topk_idx


# speedup vs baseline: 1.1108x; 1.0200x over previous
import jax
import jax.numpy as jnp
from jax.experimental import pallas as pl
from jax.experimental.pallas import tpu as pltpu

_B, _C, _H = 8, 384, 224
_R, _E, _K = 24, 8, 2
_HO = _H // 2
_NPIX = _HO * _HO
_CB = 32
_NCB = _C // _CB
_NF = _HO * _H


def _stage1_kernel(x_ref, w_ref, pw_ref, fwt_ref, fb_ref,
                   wout_ref, iout_ref, acc_ref, pool_ref):
    b = pl.program_id(0)
    c = pl.program_id(1)

    @pl.when(c == 0)
    def _init():
        acc_ref[...] = jnp.zeros_like(acc_ref)

    xb = x_ref[0]
    w = w_ref[...]

    def wcol(k):
        return w[:, k][:, None].astype(jnp.bfloat16)

    x4 = xb.reshape(_CB, _H // 8, 8, _H)
    u = jax.lax.broadcasted_iota(jnp.int32, (_CB, _H // 8, 8, _H), 2)
    gidx = ((u & 3) << 1) | (u >> 2)
    xp = jnp.take_along_axis(x4, gidx, axis=2)
    ev = xp[:, :, 0:4, :].reshape(_CB, _NF).astype(jnp.bfloat16)
    od = xp[:, :, 4:8, :].reshape(_CB, _NF).astype(jnp.bfloat16)

    zrow = jnp.zeros((_CB, _H), jnp.bfloat16)
    zone = jnp.zeros((_CB, 1), jnp.bfloat16)
    up = jnp.concatenate([zrow, od[:, :-_H]], axis=1)

    a0 = wcol(0) * up + wcol(3) * ev + wcol(6) * od
    a1 = wcol(1) * up + wcol(4) * ev + wcol(7) * od
    a2 = wcol(2) * up + wcol(5) * ev + wcol(8) * od

    s = jax.lax.broadcasted_iota(jnp.int32, (1, _NF), 1) % _H
    m0 = (s != 0).astype(jnp.bfloat16)

    y = (m0 * jnp.concatenate([zone, a0[:, :-1]], axis=1)
         + a1
         + jnp.concatenate([a2[:, 1:], zone], axis=1)
         + wcol(9))
    y = (0.5 * y) * (1.0 + jnp.tanh(0.5 * y))
    acc_ref[...] += jnp.dot(pw_ref[0].astype(jnp.bfloat16), y,
                            preferred_element_type=jnp.float32)

    @pl.when(c == _NCB - 1)
    def _finish():
        z = acc_ref[...]
        z = z * jax.nn.sigmoid(z)
        p = jax.lax.broadcasted_iota(jnp.int32, (1, _NF), 1)
        pool = ((p % 2) == 0).astype(jnp.float32)
        pool_ref[b, :] = jnp.sum(z * pool, axis=1) * (1.0 / _NPIX)

    @pl.when((b == _B - 1) & (c == _NCB - 1))
    def _head():
        pp = pool_ref[...]
        logits = jnp.dot(pp, fwt_ref[...],
                         preferred_element_type=jnp.float32) + fb_ref[...]
        mx = jnp.max(logits, axis=1, keepdims=True)
        e = jnp.exp(logits - mx)
        probs = e / jnp.sum(e, axis=1, keepdims=True)
        colid = jax.lax.broadcasted_iota(jnp.int32, (_B, _E), 1)
        i1 = jnp.min(jnp.where(logits == mx, colid, _E), axis=1,
                     keepdims=True)
        masked = jnp.where(colid == i1, -jnp.inf, logits)
        v2 = jnp.max(masked, axis=1, keepdims=True)
        i2 = jnp.min(jnp.where(masked == v2, colid, _E), axis=1,
                     keepdims=True)
        s1 = jnp.sum(jnp.where(colid == i1, probs, 0.0), axis=1,
                     keepdims=True)
        s2 = jnp.sum(jnp.where(colid == i2, probs, 0.0), axis=1,
                     keepdims=True)
        tot = s1 + s2 + 1e-6
        wout_ref[...] = jnp.concatenate([s1 / tot, s2 / tot], axis=1)
        iout_ref[...] = jnp.concatenate([i1, i2], axis=1)


def kernel(x, dw_w, bn_gamma, bn_beta, bn_mean, bn_var, pw_w, fc_w, fc_b):
    scale = bn_gamma * jax.lax.rsqrt(bn_var + 1e-5)
    bias = bn_beta - bn_mean * scale
    w9 = dw_w.reshape(_C, 9) * scale[:, None]
    wpack = jnp.concatenate(
        [w9, bias[:, None], jnp.zeros((_C, 6), jnp.float32)], axis=1)
    pw = pw_w.reshape(_R, _NCB, _CB).transpose(1, 0, 2)

    call = pl.pallas_call(
        _stage1_kernel,
        grid=(_B, _NCB),
        in_specs=[
            pl.BlockSpec((1, _CB, _H, _H), lambda b, c: (b, c, 0, 0)),
            pl.BlockSpec((_CB, 16), lambda b, c: (c, 0)),
            pl.BlockSpec((1, _R, _CB), lambda b, c: (c, 0, 0)),
            pl.BlockSpec((_R, _E), lambda b, c: (0, 0)),
            pl.BlockSpec((1, _E), lambda b, c: (0, 0)),
        ],
        out_specs=(pl.BlockSpec((_B, _K), lambda b, c: (0, 0)),
                   pl.BlockSpec((_B, _K), lambda b, c: (0, 0))),
        out_shape=(jax.ShapeDtypeStruct((_B, _K), jnp.float32),
                   jax.ShapeDtypeStruct((_B, _K), jnp.int32)),
        scratch_shapes=[pltpu.VMEM((_R, _NF), jnp.float32),
                        pltpu.VMEM((_B, _R), jnp.float32)],
        compiler_params=pltpu.CompilerParams(
            dimension_semantics=("arbitrary", "arbitrary")),
    )
    weights, topk_idx = call(x, wpack, pw, fc_w.T, fc_b.reshape(1, _E))
    return weights, topk_idx


# speedup vs baseline: 1.1113x; 1.0004x over previous
import jax
import jax.numpy as jnp
from jax.experimental import pallas as pl
from jax.experimental.pallas import tpu as pltpu

_B, _C, _H = 8, 384, 224
_R, _E, _K = 24, 8, 2
_HO = _H // 2
_NPIX = _HO * _HO
_CB = 32
_NCB = _C // _CB
_NF = _HO * _H


def _stage1_kernel(x_ref, w_ref, pw_ref, fwt_ref, fb_ref,
                   wout_ref, iout_ref, acc_ref, pool_ref):
    b = pl.program_id(0)
    c = pl.program_id(1)

    @pl.when(c == 0)
    def _init():
        acc_ref[...] = jnp.zeros_like(acc_ref)

    xb = x_ref[0]
    w = w_ref[...]

    def wcol(k):
        return w[:, k][:, None].astype(jnp.bfloat16)

    x4 = xb.reshape(_CB, _H // 8, 8, _H)
    u = jax.lax.broadcasted_iota(jnp.int32, (_CB, _H // 8, 8, _H), 2)
    gidx = ((u & 3) << 1) | (u >> 2)
    xp = jnp.take_along_axis(x4, gidx, axis=2)
    ev = xp[:, :, 0:4, :].reshape(_CB, _NF).astype(jnp.bfloat16)
    od = xp[:, :, 4:8, :].reshape(_CB, _NF).astype(jnp.bfloat16)

    zrow = jnp.zeros((_CB, _H), jnp.bfloat16)
    zone = jnp.zeros((_CB, 1), jnp.bfloat16)
    up = jnp.concatenate([zrow, od[:, :-_H]], axis=1)

    a0 = wcol(0) * up + wcol(3) * ev + wcol(6) * od
    a1 = wcol(1) * up + wcol(4) * ev + wcol(7) * od
    a2 = wcol(2) * up + wcol(5) * ev + wcol(8) * od

    s = jax.lax.broadcasted_iota(jnp.int32, (1, _NF), 1) % _H
    m0 = (s != 0).astype(jnp.bfloat16)

    y = (m0 * jnp.concatenate([zone, a0[:, :-1]], axis=1)
         + a1
         + jnp.concatenate([a2[:, 1:], zone], axis=1)
         + wcol(9))
    y = (0.5 * y) * (1.0 + jnp.tanh(0.5 * y))
    acc_ref[...] += jnp.dot(pw_ref[0].astype(jnp.bfloat16), y,
                            preferred_element_type=jnp.float32)

    @pl.when(c == _NCB - 1)
    def _finish():
        z = acc_ref[...]
        z = (0.5 * z) * (1.0 + jnp.tanh(0.5 * z))
        p = jax.lax.broadcasted_iota(jnp.int32, (1, _NF), 1)
        pool = ((p % 2) == 0).astype(jnp.float32)
        pool_ref[b, :] = jnp.sum(z * pool, axis=1) * (1.0 / _NPIX)

    @pl.when((b == _B - 1) & (c == _NCB - 1))
    def _head():
        pp = pool_ref[...]
        logits = jnp.dot(pp, fwt_ref[...],
                         preferred_element_type=jnp.float32) + fb_ref[...]
        mx = jnp.max(logits, axis=1, keepdims=True)
        e = jnp.exp(logits - mx)
        probs = e / jnp.sum(e, axis=1, keepdims=True)
        colid = jax.lax.broadcasted_iota(jnp.int32, (_B, _E), 1)
        i1 = jnp.min(jnp.where(logits == mx, colid, _E), axis=1,
                     keepdims=True)
        masked = jnp.where(colid == i1, -jnp.inf, logits)
        v2 = jnp.max(masked, axis=1, keepdims=True)
        i2 = jnp.min(jnp.where(masked == v2, colid, _E), axis=1,
                     keepdims=True)
        s1 = jnp.sum(jnp.where(colid == i1, probs, 0.0), axis=1,
                     keepdims=True)
        s2 = jnp.sum(jnp.where(colid == i2, probs, 0.0), axis=1,
                     keepdims=True)
        tot = s1 + s2 + 1e-6
        wout_ref[...] = jnp.concatenate([s1 / tot, s2 / tot], axis=1)
        iout_ref[...] = jnp.concatenate([i1, i2], axis=1)


def kernel(x, dw_w, bn_gamma, bn_beta, bn_mean, bn_var, pw_w, fc_w, fc_b):
    scale = bn_gamma * jax.lax.rsqrt(bn_var + 1e-5)
    bias = bn_beta - bn_mean * scale
    w9 = dw_w.reshape(_C, 9) * scale[:, None]
    wpack = jnp.concatenate(
        [w9, bias[:, None], jnp.zeros((_C, 6), jnp.float32)], axis=1)
    pw = pw_w.reshape(_R, _NCB, _CB).transpose(1, 0, 2)

    call = pl.pallas_call(
        _stage1_kernel,
        grid=(_B, _NCB),
        in_specs=[
            pl.BlockSpec((1, _CB, _H, _H), lambda b, c: (b, c, 0, 0)),
            pl.BlockSpec((_CB, 16), lambda b, c: (c, 0)),
            pl.BlockSpec((1, _R, _CB), lambda b, c: (c, 0, 0)),
            pl.BlockSpec((_R, _E), lambda b, c: (0, 0)),
            pl.BlockSpec((1, _E), lambda b, c: (0, 0)),
        ],
        out_specs=(pl.BlockSpec((_B, _K), lambda b, c: (0, 0)),
                   pl.BlockSpec((_B, _K), lambda b, c: (0, 0))),
        out_shape=(jax.ShapeDtypeStruct((_B, _K), jnp.float32),
                   jax.ShapeDtypeStruct((_B, _K), jnp.int32)),
        scratch_shapes=[pltpu.VMEM((_R, _NF), jnp.float32),
                        pltpu.VMEM((_B, _R), jnp.float32)],
        compiler_params=pltpu.CompilerParams(
            dimension_semantics=("arbitrary", "arbitrary")),
    )
    weights, topk_idx = call(x, wpack, pw, fc_w.T, fc_b.reshape(1, _E))
    return weights, topk_idx
